# baseline (speedup 1.0000x reference)
import jax
import jax.numpy as jnp
from jax.experimental import pallas as pl
from jax.experimental.pallas import tpu as pltpu

N = 16
D = 32
H = 64
BB = 512


def _body(locs_ref, states_ref, w1a_ref, w1b_ref, w1c_ref, b1_ref, w45_ref,
          b45_ref, out_ref, a0_ref):
    ft = states_ref[...].T
    lt = locs_ref[...].T
    lx = lt[:N]
    ly = lt[N:]

    degs = []
    for i in range(N):
        dx = lx[i:i + 1] - lx
        dy = ly[i:i + 1] - ly
        a0row = ((dx * dx + dy * dy) < 1.0).astype(jnp.float32)
        a0_ref[i] = a0row
        degs.append(jnp.sum(a0row, axis=0, keepdims=True))
    dinv = jax.lax.rsqrt(jnp.concatenate(degs, axis=0))

    gs = [ft[j * D:(j + 1) * D] * dinv[j:j + 1] for j in range(N)]
    glx = lx * dinv
    gly = ly * dinv

    w1a = w1a_ref[...]
    w1b = w1b_ref[...]
    w1c = w1c_ref[...]
    b1s = b1_ref[...]
    w45 = w45_ref[...]
    b45 = b45_ref[...]

    rows = []
    for i in range(N):
        a0row = a0_ref[i]
        di = dinv[i:i + 1]
        s = a0row[0:1] * gs[0]
        for j in range(1, N):
            s = s + a0row[j:j + 1] * gs[j]
        acc = di * (s - gs[i])

        t = jnp.sum(a0row * dinv, axis=0, keepdims=True)
        rs = di * t - di * di
        sx = jnp.sum(a0row * glx, axis=0, keepdims=True)
        sy = jnp.sum(a0row * gly, axis=0, keepdims=True)
        dsx = rs * lx[i:i + 1] - di * (sx - glx[i:i + 1])
        dsy = rs * ly[i:i + 1] - di * (sy - gly[i:i + 1])

        x = jnp.dot(w1a, ft[i * D:(i + 1) * D],
                    preferred_element_type=jnp.float32)
        x = x + jnp.dot(w1b, acc, preferred_element_type=jnp.float32)
        x = x + w1c[:, 0:1] * dsx + w1c[:, 1:2] * dsy + b1s
        s2 = jnp.maximum(x, 0.0)
        pv = jnp.dot(w45, s2, preferred_element_type=jnp.float32) + b45
        rows.append(pv)

    out_ref[...] = jnp.concatenate(rows, axis=0).T


@jax.jit
def kernel(locs, states, W1, b1, W4, b4, W5, b5):
    B = locs.shape[0]
    G = B // BB

    locs2 = locs.transpose(0, 2, 1).reshape(B, 2 * N)
    states2 = states.reshape(B, N * D)
    w1a = (N - 1.0) * W1[:D].T
    w1b = W1[D:2 * D].T
    w1c = W1[2 * D:].T
    b1s = ((N - 1.0) * b1)[:, None]
    w45 = jnp.concatenate([W4, W5], axis=1).T
    b45 = jnp.concatenate([b4, b5], axis=0)[:, None]

    out = pl.pallas_call(
        _body,
        grid=(G,),
        in_specs=[
            pl.BlockSpec((BB, 2 * N), lambda g: (g, 0)),
            pl.BlockSpec((BB, N * D), lambda g: (g, 0)),
            pl.BlockSpec((H, D), lambda g: (0, 0)),
            pl.BlockSpec((H, D), lambda g: (0, 0)),
            pl.BlockSpec((H, 2), lambda g: (0, 0)),
            pl.BlockSpec((H, 1), lambda g: (0, 0)),
            pl.BlockSpec((3, H), lambda g: (0, 0)),
            pl.BlockSpec((3, 1), lambda g: (0, 0)),
        ],
        out_specs=pl.BlockSpec((BB, N * 3), lambda g: (g, 0)),
        out_shape=jax.ShapeDtypeStruct((B, N * 3), jnp.float32),
        scratch_shapes=[pltpu.VMEM((N, N, BB), jnp.float32)],
    )(locs2, states2, w1a, w1b, w1c, b1s, w45, b45)

    pv = out.reshape(B, N, 3)
    return pv[:, :, :2], pv[:, :, 2:]


# speedup vs baseline: 1.0022x; 1.0022x over previous
import jax
import jax.numpy as jnp
from jax import lax
from jax.experimental import pallas as pl
from jax.experimental.pallas import tpu as pltpu

N = 16
D = 32
H = 64
BB = 512

_C00 = (((0,), (0,)), ((), ()))


def _body(locs_ref, states_ref, w1_ref, b1_ref, w4_ref, w5_ref, b4_ref,
          b5_ref, pol_ref, val_ref, a0_ref):
    ft = states_ref[...].T
    lti = locs_ref[...].T

    r = lax.broadcasted_iota(jnp.int32, (2 * N, 2 * N), 0)
    s = lax.broadcasted_iota(jnp.int32, (2 * N, 2 * N), 1)
    perm = (s == 2 * (r % N) + r // N).astype(jnp.float32)
    lt = jnp.dot(perm, lti, preferred_element_type=jnp.float32)
    lx = lt[:N]
    ly = lt[N:]

    degs = []
    for i in range(N):
        dx = lx[i:i + 1] - lx
        dy = ly[i:i + 1] - ly
        a0row = ((dx * dx + dy * dy) < 1.0).astype(jnp.float32)
        a0_ref[i] = a0row
        degs.append(jnp.sum(a0row, axis=0, keepdims=True))
    dinv = lax.rsqrt(jnp.concatenate(degs, axis=0))

    gs = [ft[j * D:(j + 1) * D] * dinv[j:j + 1] for j in range(N)]
    glx = lx * dinv
    gly = ly * dinv

    w1 = w1_ref[...]
    w1a15 = w1[:D] * (N - 1.0)
    w1b = w1[D:2 * D]
    w1cb = jnp.concatenate([w1[2 * D:], (N - 1.0) * b1_ref[...]], axis=0)
    w45b = jnp.concatenate([
        jnp.concatenate([w4_ref[...], w5_ref[...]], axis=1),
        jnp.concatenate([b4_ref[...], b5_ref[...]], axis=1),
    ], axis=0)
    ones = jnp.ones((1, BB), jnp.float32)

    rows_p = []
    rows_v = []
    for i in range(N):
        a0row = a0_ref[i]
        di = dinv[i:i + 1]
        agg = a0row[0:1] * gs[0]
        for j in range(1, N):
            agg = agg + a0row[j:j + 1] * gs[j]
        acc = di * (agg - gs[i])

        t = jnp.sum(a0row * dinv, axis=0, keepdims=True)
        rs = di * t - di * di
        sx = jnp.sum(a0row * glx, axis=0, keepdims=True)
        sy = jnp.sum(a0row * gly, axis=0, keepdims=True)
        dsx = rs * lx[i:i + 1] - di * (sx - glx[i:i + 1])
        dsy = rs * ly[i:i + 1] - di * (sy - gly[i:i + 1])

        x = lax.dot_general(w1a15, ft[i * D:(i + 1) * D], _C00,
                            preferred_element_type=jnp.float32)
        x = x + lax.dot_general(w1b, acc, _C00,
                                preferred_element_type=jnp.float32)
        dse = jnp.concatenate([dsx, dsy, ones], axis=0)
        x = x + lax.dot_general(w1cb, dse, _C00,
                                preferred_element_type=jnp.float32)
        s2 = jnp.maximum(x, 0.0)
        s2e = jnp.concatenate([s2, ones], axis=0)
        pv = lax.dot_general(w45b, s2e, _C00,
                             preferred_element_type=jnp.float32)
        rows_p.append(pv[0:2])
        rows_v.append(pv[2:3])

    pol_ref[...] = jnp.concatenate(rows_p, axis=0).T
    val_ref[...] = jnp.concatenate(rows_v, axis=0).T


@jax.jit
def kernel(locs, states, W1, b1, W4, b4, W5, b5):
    B = locs.shape[0]
    G = B // BB

    pol, val = pl.pallas_call(
        _body,
        grid=(G,),
        in_specs=[
            pl.BlockSpec((BB, 2 * N), lambda g: (g, 0)),
            pl.BlockSpec((BB, N * D), lambda g: (g, 0)),
            pl.BlockSpec((2 * D + 2, H), lambda g: (0, 0)),
            pl.BlockSpec((1, H), lambda g: (0, 0)),
            pl.BlockSpec((H, 2), lambda g: (0, 0)),
            pl.BlockSpec((H, 1), lambda g: (0, 0)),
            pl.BlockSpec((1, 2), lambda g: (0, 0)),
            pl.BlockSpec((1, 1), lambda g: (0, 0)),
        ],
        out_specs=[
            pl.BlockSpec((BB, N * 2), lambda g: (g, 0)),
            pl.BlockSpec((BB, N), lambda g: (g, 0)),
        ],
        out_shape=[
            jax.ShapeDtypeStruct((B, N * 2), jnp.float32),
            jax.ShapeDtypeStruct((B, N), jnp.float32),
        ],
        scratch_shapes=[pltpu.VMEM((N, N, BB), jnp.float32)],
    )(locs.reshape(B, 2 * N), states.reshape(B, N * D), W1, b1[None, :],
      W4, W5, b4[None, :], b5[None, :])

    return pol.reshape(B, N, 2), val.reshape(B, N, 1)


# speedup vs baseline: 1.1343x; 1.1318x over previous
import jax
import jax.numpy as jnp
from jax import lax
from jax.experimental import pallas as pl
from jax.experimental.pallas import tpu as pltpu

N = 16
D = 32
H = 64
BB = 1024

_C00 = (((0,), (0,)), ((), ()))


def _body(locs_ref, states_ref, w1_ref, b1_ref, w4_ref, w5_ref, b4_ref,
          b5_ref, pol_ref, val_ref, a0_ref):
    ft = states_ref[...].T
    lti = locs_ref[...].T

    r = lax.broadcasted_iota(jnp.int32, (2 * N, 2 * N), 0)
    s = lax.broadcasted_iota(jnp.int32, (2 * N, 2 * N), 1)
    perm = (s == 2 * (r % N) + r // N).astype(jnp.float32)
    lt = jnp.dot(perm, lti, preferred_element_type=jnp.float32)
    lx = lt[:N]
    ly = lt[N:]

    degs = []
    for i in range(N):
        dx = lx[i:i + 1] - lx
        dy = ly[i:i + 1] - ly
        a0row = ((dx * dx + dy * dy) < 1.0).astype(jnp.float32)
        a0_ref[i] = a0row
        degs.append(jnp.sum(a0row, axis=0, keepdims=True))
    dinv = lax.rsqrt(jnp.concatenate(degs, axis=0))

    gs = [ft[j * D:(j + 1) * D] * dinv[j:j + 1] for j in range(N)]
    glx = lx * dinv
    gly = ly * dinv

    w1 = w1_ref[...]
    w1a15 = w1[:D] * (N - 1.0)
    w1b = w1[D:2 * D]
    w1cb = jnp.concatenate([w1[2 * D:], (N - 1.0) * b1_ref[...]], axis=0)
    w45b = jnp.concatenate([
        jnp.concatenate([w4_ref[...], w5_ref[...]], axis=1),
        jnp.concatenate([b4_ref[...], b5_ref[...]], axis=1),
    ], axis=0)
    ones = jnp.ones((1, BB), jnp.float32)

    rows_p = []
    rows_v = []
    for i in range(N):
        a0row = a0_ref[i]
        di = dinv[i:i + 1]
        agg = a0row[0:1] * gs[0]
        for j in range(1, N):
            agg = agg + a0row[j:j + 1] * gs[j]
        acc = di * (agg - gs[i])

        t = jnp.sum(a0row * dinv, axis=0, keepdims=True)
        rs = di * t - di * di
        sx = jnp.sum(a0row * glx, axis=0, keepdims=True)
        sy = jnp.sum(a0row * gly, axis=0, keepdims=True)
        dsx = rs * lx[i:i + 1] - di * (sx - glx[i:i + 1])
        dsy = rs * ly[i:i + 1] - di * (sy - gly[i:i + 1])

        x = lax.dot_general(w1a15, ft[i * D:(i + 1) * D], _C00,
                            preferred_element_type=jnp.float32)
        x = x + lax.dot_general(w1b, acc, _C00,
                                preferred_element_type=jnp.float32)
        dse = jnp.concatenate([dsx, dsy, ones], axis=0)
        x = x + lax.dot_general(w1cb, dse, _C00,
                                preferred_element_type=jnp.float32)
        s2 = jnp.maximum(x, 0.0)
        s2e = jnp.concatenate([s2, ones], axis=0)
        pv = lax.dot_general(w45b, s2e, _C00,
                             preferred_element_type=jnp.float32)
        rows_p.append(pv[0:2])
        rows_v.append(pv[2:3])

    pol_ref[...] = jnp.concatenate(rows_p, axis=0).T
    val_ref[...] = jnp.concatenate(rows_v, axis=0).T


@jax.jit
def kernel(locs, states, W1, b1, W4, b4, W5, b5):
    B = locs.shape[0]
    G = B // BB

    pol, val = pl.pallas_call(
        _body,
        grid=(G,),
        in_specs=[
            pl.BlockSpec((BB, 2 * N), lambda g: (g, 0)),
            pl.BlockSpec((BB, N * D), lambda g: (g, 0)),
            pl.BlockSpec((2 * D + 2, H), lambda g: (0, 0)),
            pl.BlockSpec((1, H), lambda g: (0, 0)),
            pl.BlockSpec((H, 2), lambda g: (0, 0)),
            pl.BlockSpec((H, 1), lambda g: (0, 0)),
            pl.BlockSpec((1, 2), lambda g: (0, 0)),
            pl.BlockSpec((1, 1), lambda g: (0, 0)),
        ],
        out_specs=[
            pl.BlockSpec((BB, N * 2), lambda g: (g, 0)),
            pl.BlockSpec((BB, N), lambda g: (g, 0)),
        ],
        out_shape=[
            jax.ShapeDtypeStruct((B, N * 2), jnp.float32),
            jax.ShapeDtypeStruct((B, N), jnp.float32),
        ],
        scratch_shapes=[pltpu.VMEM((N, N, BB), jnp.float32)],
    )(locs.reshape(B, 2 * N), states.reshape(B, N * D), W1, b1[None, :],
      W4, W5, b4[None, :], b5[None, :])

    return pol.reshape(B, N, 2), val.reshape(B, N, 1)


# speedup vs baseline: 1.1472x; 1.0114x over previous
import jax
import jax.numpy as jnp
from jax import lax
from jax.experimental import pallas as pl
from jax.experimental.pallas import tpu as pltpu

N = 16
D = 32
H = 64
BB = 2048

_C00 = (((0,), (0,)), ((), ()))


def _body(locs_ref, states_ref, w1_ref, b1_ref, w4_ref, w5_ref, b4_ref,
          b5_ref, pol_ref, val_ref, a0_ref):
    ft = states_ref[...].T
    lti = locs_ref[...].T

    r = lax.broadcasted_iota(jnp.int32, (2 * N, 2 * N), 0)
    s = lax.broadcasted_iota(jnp.int32, (2 * N, 2 * N), 1)
    perm = (s == 2 * (r % N) + r // N).astype(jnp.float32)
    lt = jnp.dot(perm, lti, preferred_element_type=jnp.float32)
    lx = lt[:N]
    ly = lt[N:]

    degs = []
    for i in range(N):
        dx = lx[i:i + 1] - lx
        dy = ly[i:i + 1] - ly
        a0row = ((dx * dx + dy * dy) < 1.0).astype(jnp.float32)
        a0_ref[i] = a0row
        degs.append(jnp.sum(a0row, axis=0, keepdims=True))
    dinv = lax.rsqrt(jnp.concatenate(degs, axis=0))

    gs = [ft[j * D:(j + 1) * D] * dinv[j:j + 1] for j in range(N)]
    glx = lx * dinv
    gly = ly * dinv

    w1 = w1_ref[...]
    w1a15 = w1[:D] * (N - 1.0)
    w1b = w1[D:2 * D]
    w1cb = jnp.concatenate([w1[2 * D:], (N - 1.0) * b1_ref[...]], axis=0)
    w45b = jnp.concatenate([
        jnp.concatenate([w4_ref[...], w5_ref[...]], axis=1),
        jnp.concatenate([b4_ref[...], b5_ref[...]], axis=1),
    ], axis=0)
    ones = jnp.ones((1, BB), jnp.float32)

    rows_p = []
    rows_v = []
    for i in range(N):
        a0row = a0_ref[i]
        di = dinv[i:i + 1]
        agg = a0row[0:1] * gs[0]
        for j in range(1, N):
            agg = agg + a0row[j:j + 1] * gs[j]
        acc = di * (agg - gs[i])

        t = jnp.sum(a0row * dinv, axis=0, keepdims=True)
        rs = di * t - di * di
        sx = jnp.sum(a0row * glx, axis=0, keepdims=True)
        sy = jnp.sum(a0row * gly, axis=0, keepdims=True)
        dsx = rs * lx[i:i + 1] - di * (sx - glx[i:i + 1])
        dsy = rs * ly[i:i + 1] - di * (sy - gly[i:i + 1])

        x = lax.dot_general(w1a15, ft[i * D:(i + 1) * D], _C00,
                            preferred_element_type=jnp.float32)
        x = x + lax.dot_general(w1b, acc, _C00,
                                preferred_element_type=jnp.float32)
        dse = jnp.concatenate([dsx, dsy, ones], axis=0)
        x = x + lax.dot_general(w1cb, dse, _C00,
                                preferred_element_type=jnp.float32)
        s2 = jnp.maximum(x, 0.0)
        s2e = jnp.concatenate([s2, ones], axis=0)
        pv = lax.dot_general(w45b, s2e, _C00,
                             preferred_element_type=jnp.float32)
        rows_p.append(pv[0:2])
        rows_v.append(pv[2:3])

    pol_ref[...] = jnp.concatenate(rows_p, axis=0).T
    val_ref[...] = jnp.concatenate(rows_v, axis=0).T


@jax.jit
def kernel(locs, states, W1, b1, W4, b4, W5, b5):
    B = locs.shape[0]
    G = B // BB

    pol, val = pl.pallas_call(
        _body,
        grid=(G,),
        in_specs=[
            pl.BlockSpec((BB, 2 * N), lambda g: (g, 0)),
            pl.BlockSpec((BB, N * D), lambda g: (g, 0)),
            pl.BlockSpec((2 * D + 2, H), lambda g: (0, 0)),
            pl.BlockSpec((1, H), lambda g: (0, 0)),
            pl.BlockSpec((H, 2), lambda g: (0, 0)),
            pl.BlockSpec((H, 1), lambda g: (0, 0)),
            pl.BlockSpec((1, 2), lambda g: (0, 0)),
            pl.BlockSpec((1, 1), lambda g: (0, 0)),
        ],
        out_specs=[
            pl.BlockSpec((BB, N * 2), lambda g: (g, 0)),
            pl.BlockSpec((BB, N), lambda g: (g, 0)),
        ],
        out_shape=[
            jax.ShapeDtypeStruct((B, N * 2), jnp.float32),
            jax.ShapeDtypeStruct((B, N), jnp.float32),
        ],
        scratch_shapes=[pltpu.VMEM((N, N, BB), jnp.float32)],
    )(locs.reshape(B, 2 * N), states.reshape(B, N * D), W1, b1[None, :],
      W4, W5, b4[None, :], b5[None, :])

    return pol.reshape(B, N, 2), val.reshape(B, N, 1)
